# Initial kernel scaffold; baseline (speedup 1.0000x reference)
import jax
import jax.numpy as jnp
from jax.experimental import pallas as pl


def kernel(ids, codebook, W_proj, b_proj, W_out, b_out, W_dec, b_dec):
    raise NotImplementedError("write your pallas kernel here")


# speedup vs baseline: 2.1684x; 2.1684x over previous
import functools

import jax
import jax.numpy as jnp
from jax import lax
from jax.experimental import pallas as pl
from jax.experimental.pallas import tpu as pltpu
from jax.experimental.pallas import tpu_sc as plsc

K_ROWS = 8192
CODE_DIM = 256
EMBED_DIM = 512
OUT_DIM = 512
N_TOKENS = 8192

NC, NS = 2, 16
NW = NC * NS
B_PER_W = N_TOKENS // NW
IDX_CHUNK = 128


def _gather_body(table_hbm, idx_hbm, out_hbm, idx_v, rows_v, sem):
    wid = lax.axis_index("s") * NC + lax.axis_index("c")
    base = wid * B_PER_W
    pltpu.sync_copy(idx_hbm.at[pl.ds(base, B_PER_W)], idx_v)
    copies = []
    for j in range(B_PER_W // IDX_CHUNK):
        copies.append(
            pltpu.async_copy(
                table_hbm.at[idx_v.at[pl.ds(j * IDX_CHUNK, IDX_CHUNK)]],
                rows_v.at[pl.ds(j * IDX_CHUNK, IDX_CHUNK)],
                sem,
            )
        )
    for c in copies:
        c.wait()
    pltpu.sync_copy(rows_v, out_hbm.at[pl.ds(base, B_PER_W)])


@functools.cache
def _sc_gather_fn():
    return pl.kernel(
        _gather_body,
        out_type=jax.ShapeDtypeStruct((N_TOKENS, CODE_DIM), jnp.float32),
        mesh=plsc.VectorSubcoreMesh(core_axis_name="c", subcore_axis_name="s"),
        scratch_types=[
            pltpu.VMEM((B_PER_W,), jnp.int32),
            pltpu.VMEM((B_PER_W, CODE_DIM), jnp.float32),
            pltpu.SemaphoreType.DMA,
        ],
    )


M_BLK = 1024


def _mm_body(emb_ref, wp_ref, wo_ref, wd_ref, bp_ref, bo_ref, bd_ref,
             out_ref, wf_ref, bf_ref):
    @pl.when(pl.program_id(0) == 0)
    def _fuse_weights():
        t = jnp.dot(wp_ref[...], wo_ref[...],
                    preferred_element_type=jnp.float32)
        wf_ref[...] = jnp.dot(t, wd_ref[...],
                              preferred_element_type=jnp.float32)
        tb = jnp.dot(bp_ref[...], wo_ref[...],
                     preferred_element_type=jnp.float32) + bo_ref[...]
        bf_ref[...] = jnp.dot(tb, wd_ref[...],
                              preferred_element_type=jnp.float32) + bd_ref[...]

    out_ref[...] = jnp.dot(emb_ref[...], wf_ref[...],
                           preferred_element_type=jnp.float32) + bf_ref[...]


def _tc_decode(emb, W_proj, b_proj, W_out, b_out, W_dec, b_dec):
    grid = (N_TOKENS // M_BLK,)
    return pl.pallas_call(
        _mm_body,
        grid=grid,
        in_specs=[
            pl.BlockSpec((M_BLK, CODE_DIM), lambda i: (i, 0)),
            pl.BlockSpec((CODE_DIM, EMBED_DIM), lambda i: (0, 0)),
            pl.BlockSpec((EMBED_DIM, EMBED_DIM), lambda i: (0, 0)),
            pl.BlockSpec((EMBED_DIM, OUT_DIM), lambda i: (0, 0)),
            pl.BlockSpec((1, EMBED_DIM), lambda i: (0, 0)),
            pl.BlockSpec((1, EMBED_DIM), lambda i: (0, 0)),
            pl.BlockSpec((1, OUT_DIM), lambda i: (0, 0)),
        ],
        out_specs=pl.BlockSpec((M_BLK, OUT_DIM), lambda i: (i, 0)),
        out_shape=jax.ShapeDtypeStruct((N_TOKENS, OUT_DIM), jnp.float32),
        scratch_shapes=[
            pltpu.VMEM((CODE_DIM, OUT_DIM), jnp.float32),
            pltpu.VMEM((1, OUT_DIM), jnp.float32),
        ],
    )(emb, W_proj, W_out, W_dec,
      b_proj.reshape(1, EMBED_DIM), b_out.reshape(1, EMBED_DIM),
      b_dec.reshape(1, OUT_DIM))


@jax.jit
def kernel(ids, codebook, W_proj, b_proj, W_out, b_out, W_dec, b_dec):
    B, T = ids.shape
    flat_ids = ids.reshape(-1).astype(jnp.int32)
    emb = _sc_gather_fn()(codebook, flat_ids)
    dec = _tc_decode(emb, W_proj, b_proj, W_out, b_out, W_dec, b_dec)
    return dec.reshape(B, T, OUT_DIM)
